# baseline (speedup 1.0000x reference)
import jax
import jax.numpy as jnp
from jax.experimental import pallas as pl
from jax.experimental.pallas import tpu as pltpu

_NUM_EMBEDDINGS = 1024
_DIM = 64
_COMMITMENT = 0.25
_TILE = 1024


def _vq_tile_kernel(x_ref, e_ref, et_ref, q_ref, idx_ref, loss_ref):
    x = x_ref[...]
    e = e_ref[...]
    et = et_ref[...]

    x2 = jnp.sum(x * x, axis=1, keepdims=True)
    e2 = jnp.sum(et * et, axis=0, keepdims=True)
    m = jax.lax.dot_general(x, et, (((1,), (0,)), ((), ())),
                            preferred_element_type=jnp.float32)
    d = (x2 + e2) - 2.0 * m

    dmin = jnp.min(d, axis=1, keepdims=True)
    ids = jax.lax.broadcasted_iota(jnp.int32, d.shape, 1)
    idx = jnp.min(jnp.where(d == dmin, ids, _NUM_EMBEDDINGS), axis=1)

    onehot = (ids == idx[:, None]).astype(jnp.float32)
    q = jax.lax.dot_general(onehot, e, (((1,), (0,)), ((), ())),
                            preferred_element_type=jnp.float32)

    diff = q - x
    part = jnp.sum(diff * diff)

    q_ref[...] = x + (q - x)
    idx_ref[...] = idx

    @pl.when(pl.program_id(0) == 0)
    def _init():
        loss_ref[0, 0] = 0.0

    loss_ref[0, 0] += part


def kernel(x, embedding):
    tokens = x.shape[0] * x.shape[1]
    flat_x = x.reshape(tokens, _DIM)
    et = embedding.T
    grid = tokens // _TILE

    q, idx, loss_sum = pl.pallas_call(
        _vq_tile_kernel,
        grid=(grid,),
        in_specs=[
            pl.BlockSpec((_TILE, _DIM), lambda i: (i, 0)),
            pl.BlockSpec((_NUM_EMBEDDINGS, _DIM), lambda i: (0, 0)),
            pl.BlockSpec((_DIM, _NUM_EMBEDDINGS), lambda i: (0, 0)),
        ],
        out_specs=[
            pl.BlockSpec((_TILE, _DIM), lambda i: (i, 0)),
            pl.BlockSpec((_TILE,), lambda i: (i,)),
            pl.BlockSpec(memory_space=pltpu.SMEM, block_shape=(1, 1),
                         index_map=lambda i: (0, 0)),
        ],
        out_shape=[
            jax.ShapeDtypeStruct((tokens, _DIM), jnp.float32),
            jax.ShapeDtypeStruct((tokens,), jnp.int32),
            jax.ShapeDtypeStruct((1, 1), jnp.float32),
        ],
    )(flat_x, embedding, et)

    mean_sq = loss_sum[0, 0] / (tokens * _DIM)
    loss = mean_sq + _COMMITMENT * mean_sq
    return (q.reshape(x.shape), loss, idx)


# speedup vs baseline: 1.0911x; 1.0911x over previous
import jax
import jax.numpy as jnp
from jax.experimental import pallas as pl
from jax.experimental.pallas import tpu as pltpu

_NUM_EMBEDDINGS = 1024
_DIM = 64
_COMMITMENT = 0.25
_TILE = 2048


def _vq_tile_kernel(x_ref, e_ref, net_ref, q_ref, idx_ref, loss_ref):
    x = x_ref[...]
    e = e_ref[...]
    net = net_ref[...]

    x2 = jnp.sum(x * x, axis=1, keepdims=True)
    e2 = 0.25 * jnp.sum(net * net, axis=0, keepdims=True)
    m2 = jax.lax.dot_general(x, net, (((1,), (0,)), ((), ())),
                             preferred_element_type=jnp.float32)
    d = (x2 + e2) + m2

    dmin = jnp.min(d, axis=1, keepdims=True)
    ids = jax.lax.broadcasted_iota(jnp.int32, d.shape, 1)
    idx = jnp.min(jnp.where(d == dmin, ids, _NUM_EMBEDDINGS),
                  axis=1, keepdims=True)

    onehot = (ids == idx).astype(jnp.float32)
    q = jax.lax.dot_general(onehot, e, (((1,), (0,)), ((), ())),
                            preferred_element_type=jnp.float32)

    diff = q - x
    part = jnp.sum(diff * diff)

    q_ref[...] = x + (q - x)
    idx_ref[...] = idx

    @pl.when(pl.program_id(0) == 0)
    def _init():
        loss_ref[0, 0] = 0.0

    loss_ref[0, 0] += part


def kernel(x, embedding):
    tokens = x.shape[0] * x.shape[1]
    flat_x = x.reshape(tokens, _DIM)
    net = -2.0 * embedding.T
    grid = tokens // _TILE

    q, idx, loss_sum = pl.pallas_call(
        _vq_tile_kernel,
        grid=(grid,),
        in_specs=[
            pl.BlockSpec((_TILE, _DIM), lambda i: (i, 0)),
            pl.BlockSpec((_NUM_EMBEDDINGS, _DIM), lambda i: (0, 0)),
            pl.BlockSpec((_DIM, _NUM_EMBEDDINGS), lambda i: (0, 0)),
        ],
        out_specs=[
            pl.BlockSpec((_TILE, _DIM), lambda i: (i, 0)),
            pl.BlockSpec((_TILE, 1), lambda i: (i, 0)),
            pl.BlockSpec(memory_space=pltpu.SMEM, block_shape=(1, 1),
                         index_map=lambda i: (0, 0)),
        ],
        out_shape=[
            jax.ShapeDtypeStruct((tokens, _DIM), jnp.float32),
            jax.ShapeDtypeStruct((tokens, 1), jnp.int32),
            jax.ShapeDtypeStruct((1, 1), jnp.float32),
        ],
    )(flat_x, embedding, net)

    mean_sq = loss_sum[0, 0] / (tokens * _DIM)
    loss = mean_sq + _COMMITMENT * mean_sq
    return (q.reshape(x.shape), loss, idx.reshape(tokens))
